# Initial kernel scaffold; baseline (speedup 1.0000x reference)
import jax
import jax.numpy as jnp
from jax.experimental import pallas as pl


def kernel(source_feat, target_feat, edge_src, W1, b1, W2, b2, Ws, bs, Wl, bl):
    raise NotImplementedError("write your pallas kernel here")


# speedup vs baseline: 2.5243x; 2.5243x over previous
import functools

import jax
import jax.numpy as jnp
from jax import lax
from jax.experimental import pallas as pl
from jax.experimental.pallas import tpu as pltpu
from jax.experimental.pallas import tpu_sc as plsc

_NUM_SC_CORES = 2
_NUM_SC_SUBCORES = 16
_NUM_WORKERS = _NUM_SC_CORES * _NUM_SC_SUBCORES
_CHUNK = 128
_TBLW = 80


def _src_table_body(src_ref, wtab_ref, out_ref):
    out_ref[...] = jnp.dot(src_ref[...], wtab_ref[...],
                           preferred_element_type=jnp.float32)


def _make_src_table(source_feat, wtab):
    ns, d = source_feat.shape
    return pl.pallas_call(
        _src_table_body,
        out_shape=jax.ShapeDtypeStruct((ns, _TBLW), jnp.float32),
    )(source_feat, wtab)


def _sc_gather(table, idx3):
    nw, nj, nc = idx3.shape
    n_out = nw * nj * nc
    mesh = plsc.VectorSubcoreMesh(
        core_axis_name="c", subcore_axis_name="s",
        num_cores=_NUM_SC_CORES, num_subcores=_NUM_SC_SUBCORES)

    @functools.partial(
        pl.kernel, mesh=mesh,
        compiler_params=pltpu.CompilerParams(use_tc_tiling_on_sc=False),
        out_type=jax.ShapeDtypeStruct((n_out, _TBLW), jnp.float32),
        scratch_types=[
            pltpu.VMEM((nj, nc), jnp.int32),
            pltpu.VMEM((nc, _TBLW), jnp.float32),
            pltpu.VMEM((nc, _TBLW), jnp.float32),
            pltpu.SemaphoreType.DMA,
            pltpu.SemaphoreType.DMA,
        ],
    )
    def gather_kernel(table_hbm, idx_hbm, out_hbm, idx_v, buf0, buf1, sem0, sem1):
        wid = lax.axis_index("s") * _NUM_SC_CORES + lax.axis_index("c")
        base = wid * (nj * nc)
        pltpu.sync_copy(idx_hbm.at[wid], idx_v)
        bufs = (buf0, buf1)
        sems = (sem0, sem1)
        cp = pltpu.async_copy(table_hbm.at[idx_v.at[0]], bufs[0], sems[0])
        for j in range(nj):
            nxt = None
            if j + 1 < nj:
                nxt = pltpu.async_copy(
                    table_hbm.at[idx_v.at[j + 1]], bufs[(j + 1) % 2], sems[(j + 1) % 2])
            cp.wait()
            pltpu.sync_copy(bufs[j % 2], out_hbm.at[pl.ds(base + j * nc, nc)])
            cp = nxt

    return gather_kernel(table, idx3)


def _finish_body(g3_ref, tf_ref, w1b_ref, b1_ref, w2_ref, wl_ref, out_ref):
    tf = tf_ref[...]
    tcb = jnp.dot(tf, w1b_ref[...],
                  preferred_element_type=jnp.float32) + b1_ref[...]
    g = g3_ref[...]
    sg = g[:, :, :64]
    sp = g[:, :, 64]
    h = jnp.maximum(sg + tcb[:, None, :], 0.0)
    s = jnp.sum(h * w2_ref[...].reshape(1, 1, 64), axis=-1)
    m = jnp.max(s, axis=-1, keepdims=True)
    e = jnp.exp(s - m)
    a = e / jnp.sum(e, axis=-1, keepdims=True)
    cross = jnp.sum(a * sp, axis=-1)
    tpred = jnp.dot(tf, wl_ref[...], preferred_element_type=jnp.float32)
    out_ref[...] = tpred + cross[:, None]


def _finish(g3, target_feat, w1b, b1, w2, wl, tb):
    nt, k, w = g3.shape
    dt = target_feat.shape[1]
    grid = nt // tb
    return pl.pallas_call(
        _finish_body,
        grid=(grid,),
        in_specs=[
            pl.BlockSpec((tb, k, w), lambda i: (i, 0, 0)),
            pl.BlockSpec((tb, dt), lambda i: (i, 0)),
            pl.BlockSpec((dt, 64), lambda i: (0, 0)),
            pl.BlockSpec((1, 64), lambda i: (0, 0)),
            pl.BlockSpec((1, 64), lambda i: (0, 0)),
            pl.BlockSpec((dt, 1), lambda i: (0, 0)),
        ],
        out_specs=pl.BlockSpec((tb, 1), lambda i: (i, 0)),
        out_shape=jax.ShapeDtypeStruct((nt, 1), jnp.float32),
    )(g3, target_feat, w1b, b1, w2, wl)


def kernel(source_feat, target_feat, edge_src, W1, b1, W2, b2, Ws, bs, Wl, bl):
    ns, d = source_feat.shape
    nt, dt = target_feat.shape
    k = edge_src.shape[1]
    w1a = W1[:d]
    w1b = W1[d:]
    wtab = jnp.concatenate(
        [w1a, Ws, jnp.zeros((d, _TBLW - d - 1), jnp.float32)], axis=1)

    table = _make_src_table(source_feat, wtab)

    n_edges = nt * k
    nj = n_edges // (_NUM_WORKERS * _CHUNK)
    idx3 = edge_src.reshape(_NUM_WORKERS, nj, _CHUNK)
    gathered = _sc_gather(table, idx3)

    g3 = gathered.reshape(nt, k, _TBLW)
    out2 = _finish(g3, target_feat, w1b, b1.reshape(1, 64), W2.reshape(1, 64),
                   Wl, tb=256)
    return out2.reshape(nt) + (bl[0] + bs[0] + 0.0 * b2[0])


# speedup vs baseline: 5.1080x; 2.0235x over previous
import functools

import jax
import jax.numpy as jnp
from jax import lax
from jax.experimental import pallas as pl
from jax.experimental.pallas import tpu as pltpu
from jax.experimental.pallas import tpu_sc as plsc

_NUM_SC_CORES = 2
_NUM_SC_SUBCORES = 16
_NUM_WORKERS = _NUM_SC_CORES * _NUM_SC_SUBCORES
_CHUNK = 128
_TBLW = 80
_K = 16
_D = 64


def _tables_body(src_ref, wts_ref, tf_ref, wtt_ref, bt_ref, tab_ref, trow_ref):
    tab_ref[...] = jnp.dot(src_ref[...], wts_ref[...],
                           preferred_element_type=jnp.float32)
    trow_ref[...] = jnp.dot(tf_ref[...], wtt_ref[...],
                            preferred_element_type=jnp.float32) + bt_ref[...]


def _make_tables(source_feat, wtab_s, target_feat, wtab_t, bias_t):
    ns = source_feat.shape[0]
    nt = target_feat.shape[0]
    return pl.pallas_call(
        _tables_body,
        out_shape=(jax.ShapeDtypeStruct((ns, _TBLW), jnp.float32),
                   jax.ShapeDtypeStruct((nt, _TBLW), jnp.float32)),
    )(source_feat, wtab_s, target_feat, wtab_t, bias_t)


def _sc_attend(table, trow, idx3, w2r):
    nw, nj, nc = idx3.shape
    nt = trow.shape[0]
    t_per_w = nt // nw
    t_per_chunk = nc // _K
    mesh = plsc.VectorSubcoreMesh(
        core_axis_name="c", subcore_axis_name="s",
        num_cores=_NUM_SC_CORES, num_subcores=_NUM_SC_SUBCORES)

    @functools.partial(
        pl.kernel, mesh=mesh,
        compiler_params=pltpu.CompilerParams(use_tc_tiling_on_sc=False,
                                             needs_layout_passes=False),
        out_type=jax.ShapeDtypeStruct((nt,), jnp.float32),
        scratch_types=[
            pltpu.VMEM((nj, nc), jnp.int32),
            pltpu.VMEM((nc, _TBLW), jnp.float32),
            pltpu.VMEM((nc, _TBLW), jnp.float32),
            pltpu.VMEM((t_per_w, _TBLW), jnp.float32),
            pltpu.VMEM((4, _K), jnp.float32),
            pltpu.VMEM((t_per_w,), jnp.float32),
            pltpu.SemaphoreType.DMA,
            pltpu.SemaphoreType.DMA,
        ],
    )
    def attend_kernel(table_hbm, trow_hbm, idx_hbm, w2_hbm, out_hbm,
                      idx_v, gbuf0, gbuf1, trow_v, w2_v, outbuf, sem0, sem1):
        wid = lax.axis_index("s") * _NUM_SC_CORES + lax.axis_index("c")
        tbase = wid * t_per_w
        pltpu.sync_copy(idx_hbm.at[wid], idx_v)
        pltpu.sync_copy(trow_hbm.at[pl.ds(tbase, t_per_w)], trow_v)
        pltpu.sync_copy(w2_hbm, w2_v)

        lane = lax.iota(jnp.int32, _K)
        in8 = lane < t_per_chunk
        col_sp = jnp.full((_K,), _D, jnp.int32)

        w2v = [w2_v[c] for c in range(4)]

        def compute_chunk(j, gbuf):
            zz = (jnp.zeros((_K,), jnp.float32), jnp.ones((_K,), jnp.float32))

            @pl.loop(0, t_per_chunk, init_carry=zz)
            def tloop(t8, carry):
                numv, denv = carry
                tglob = j * t_per_chunk + t8
                tv = [trow_v[tglob, pl.ds(c * _K, _K)] for c in range(4)]
                sv = jnp.zeros((_K,), jnp.float32)
                for k in range(_K):
                    row = t8 * _K + k
                    acc = jnp.zeros((_K,), jnp.float32)
                    for c in range(4):
                        g = gbuf[row, pl.ds(c * _K, _K)]
                        acc += jnp.maximum(g + tv[c], 0.0) * w2v[c]
                    sv = jnp.where(lane == k, jnp.sum(acc), sv)
                m = jnp.max(sv)
                ev = jnp.exp(sv - m)
                spv = plsc.load_gather(gbuf, [t8 * _K + lane, col_sp])
                numv = jnp.where(lane == t8, jnp.sum(ev * spv), numv)
                denv = jnp.where(lane == t8, jnp.sum(ev), denv)
                return numv, denv

            numv, denv = tloop
            outv = numv / denv
            tpv = plsc.load_gather(
                trow_v, [j * t_per_chunk + lane, col_sp], mask=in8)
            plsc.store_scatter(outbuf, [j * t_per_chunk + lane],
                               outv + tpv, mask=in8)

        pltpu.async_copy(table_hbm.at[idx_v.at[0]], gbuf0, sem0)
        pltpu.async_copy(table_hbm.at[idx_v.at[1]], gbuf1, sem1)

        @pl.loop(0, nj // 2)
        def jloop(i):
            ja = 2 * i
            pltpu.make_async_copy(table_hbm.at[idx_v.at[0]], gbuf0, sem0).wait()
            compute_chunk(ja, gbuf0)

            @pl.when(ja + 2 < nj)
            def _():
                pltpu.async_copy(table_hbm.at[idx_v.at[ja + 2]], gbuf0, sem0)

            pltpu.make_async_copy(table_hbm.at[idx_v.at[1]], gbuf1, sem1).wait()
            compute_chunk(ja + 1, gbuf1)

            @pl.when(ja + 3 < nj)
            def _():
                pltpu.async_copy(table_hbm.at[idx_v.at[ja + 3]], gbuf1, sem1)

        pltpu.sync_copy(outbuf, out_hbm.at[pl.ds(tbase, t_per_w)])

    return attend_kernel(table, trow, idx3, w2r)


def kernel(source_feat, target_feat, edge_src, W1, b1, W2, b2, Ws, bs, Wl, bl):
    ns, d = source_feat.shape
    nt, dt = target_feat.shape
    k = edge_src.shape[1]
    w1a = W1[:d]
    w1b = W1[d:]
    pad_s = jnp.zeros((d, _TBLW - d - 1), jnp.float32)
    pad_t = jnp.zeros((dt, _TBLW - d - 1), jnp.float32)
    wtab_s = jnp.concatenate([w1a, Ws, pad_s], axis=1)
    wtab_t = jnp.concatenate([w1b, Wl, pad_t], axis=1)
    bias_t = jnp.concatenate(
        [b1, bl + bs, jnp.zeros((_TBLW - d - 1,), jnp.float32)]).reshape(1, _TBLW)

    table, trow = _make_tables(source_feat, wtab_s, target_feat, wtab_t, bias_t)

    n_edges = nt * k
    nj = n_edges // (_NUM_WORKERS * _CHUNK)
    idx3 = edge_src.reshape(_NUM_WORKERS, nj, _CHUNK)
    w2r = W2.reshape(4, 16)
    out = _sc_attend(table, trow, idx3, w2r)
    return out + 0.0 * b2[0]


# speedup vs baseline: 5.1737x; 1.0129x over previous
import functools

import jax
import jax.numpy as jnp
from jax import lax
from jax.experimental import pallas as pl
from jax.experimental.pallas import tpu as pltpu
from jax.experimental.pallas import tpu_sc as plsc

_NUM_SC_CORES = 2
_NUM_SC_SUBCORES = 16
_NUM_WORKERS = _NUM_SC_CORES * _NUM_SC_SUBCORES
_CHUNK = 128
_NBUF = 4
_TBLW = 80
_K = 16
_D = 64


def _tables_body(src_ref, wts_ref, tf_ref, wtt_ref, bt_ref, tab_ref, trow_ref):
    tab_ref[...] = jnp.dot(src_ref[...], wts_ref[...],
                           preferred_element_type=jnp.float32)
    trow_ref[...] = jnp.dot(tf_ref[...], wtt_ref[...],
                            preferred_element_type=jnp.float32) + bt_ref[...]


def _make_tables(source_feat, wtab_s, target_feat, wtab_t, bias_t):
    ns = source_feat.shape[0]
    nt = target_feat.shape[0]
    return pl.pallas_call(
        _tables_body,
        out_shape=(jax.ShapeDtypeStruct((ns, _TBLW), jnp.float32),
                   jax.ShapeDtypeStruct((nt, _TBLW), jnp.float32)),
    )(source_feat, wtab_s, target_feat, wtab_t, bias_t)


def _sc_attend(table, trow, idx3, w2r):
    nw, nj, nc = idx3.shape
    nt = trow.shape[0]
    t_per_w = nt // nw
    t_per_chunk = nc // _K
    mesh = plsc.VectorSubcoreMesh(
        core_axis_name="c", subcore_axis_name="s",
        num_cores=_NUM_SC_CORES, num_subcores=_NUM_SC_SUBCORES)

    @functools.partial(
        pl.kernel, mesh=mesh,
        compiler_params=pltpu.CompilerParams(use_tc_tiling_on_sc=False,
                                             needs_layout_passes=False),
        out_type=jax.ShapeDtypeStruct((nt,), jnp.float32),
        scratch_types=[
            pltpu.VMEM((nj, nc), jnp.int32),
            [pltpu.VMEM((nc, _TBLW), jnp.float32) for _ in range(_NBUF)],
            pltpu.VMEM((t_per_w, _TBLW), jnp.float32),
            pltpu.VMEM((4, _K), jnp.float32),
            pltpu.VMEM((t_per_w,), jnp.float32),
            [pltpu.SemaphoreType.DMA for _ in range(_NBUF)],
            pltpu.SemaphoreType.DMA,
        ],
    )
    def attend_kernel(table_hbm, trow_hbm, idx_hbm, w2_hbm, out_hbm,
                      idx_v, gbufs, trow_v, w2_v, outbuf, sems, semt):
        wid = lax.axis_index("s") * _NUM_SC_CORES + lax.axis_index("c")
        tbase = wid * t_per_w
        pltpu.sync_copy(idx_hbm.at[wid], idx_v)
        for b in range(_NBUF):
            pltpu.async_copy(table_hbm.at[idx_v.at[b]], gbufs[b], sems[b])
        cpt = pltpu.async_copy(trow_hbm.at[pl.ds(tbase, t_per_w)], trow_v, semt)
        pltpu.sync_copy(w2_hbm, w2_v)
        cpt.wait()

        lane = lax.iota(jnp.int32, _K)
        in8 = lane < t_per_chunk
        col_sp = jnp.full((_K,), _D, jnp.int32)

        w2v = [w2_v[c] for c in range(4)]

        def compute_chunk(j, gbuf):
            zz = (jnp.zeros((_K,), jnp.float32), jnp.ones((_K,), jnp.float32))

            @pl.loop(0, t_per_chunk, init_carry=zz)
            def tloop(t8, carry):
                numv, denv = carry
                tglob = j * t_per_chunk + t8
                tv = [trow_v[tglob, pl.ds(c * _K, _K)] for c in range(4)]
                sv = jnp.zeros((_K,), jnp.float32)
                for k in range(_K):
                    row = t8 * _K + k
                    acc = jnp.zeros((_K,), jnp.float32)
                    for c in range(4):
                        g = gbuf[row, pl.ds(c * _K, _K)]
                        acc += jnp.maximum(g + tv[c], 0.0) * w2v[c]
                    sv = jnp.where(lane == k, jnp.sum(acc), sv)
                m = jnp.max(sv)
                ev = jnp.exp(sv - m)
                spv = plsc.load_gather(gbuf, [t8 * _K + lane, col_sp])
                numv = jnp.where(lane == t8, jnp.sum(ev * spv), numv)
                denv = jnp.where(lane == t8, jnp.sum(ev), denv)
                return numv, denv

            numv, denv = tloop
            outv = numv / denv
            tpv = plsc.load_gather(
                trow_v, [j * t_per_chunk + lane, col_sp], mask=in8)
            plsc.store_scatter(outbuf, [j * t_per_chunk + lane],
                               outv + tpv, mask=in8)

        @pl.loop(0, nj // _NBUF)
        def jloop(i):
            for b in range(_NBUF):
                j = _NBUF * i + b
                pltpu.make_async_copy(
                    table_hbm.at[idx_v.at[0]], gbufs[b], sems[b]).wait()
                compute_chunk(j, gbufs[b])

                @pl.when(j + _NBUF < nj)
                def _():
                    pltpu.async_copy(
                        table_hbm.at[idx_v.at[j + _NBUF]], gbufs[b], sems[b])

        pltpu.sync_copy(outbuf, out_hbm.at[pl.ds(tbase, t_per_w)])

    return attend_kernel(table, trow, idx3, w2r)


def kernel(source_feat, target_feat, edge_src, W1, b1, W2, b2, Ws, bs, Wl, bl):
    ns, d = source_feat.shape
    nt, dt = target_feat.shape
    k = edge_src.shape[1]
    w1a = W1[:d]
    w1b = W1[d:]
    pad_s = jnp.zeros((d, _TBLW - d - 1), jnp.float32)
    pad_t = jnp.zeros((dt, _TBLW - d - 1), jnp.float32)
    wtab_s = jnp.concatenate([w1a, Ws, pad_s], axis=1)
    wtab_t = jnp.concatenate([w1b, Wl, pad_t], axis=1)
    bias_t = jnp.concatenate(
        [b1, bl + bs, jnp.zeros((_TBLW - d - 1,), jnp.float32)]).reshape(1, _TBLW)

    table, trow = _make_tables(source_feat, wtab_s, target_feat, wtab_t, bias_t)

    n_edges = nt * k
    nj = n_edges // (_NUM_WORKERS * _CHUNK)
    idx3 = edge_src.reshape(_NUM_WORKERS, nj, _CHUNK)
    w2r = W2.reshape(4, 16)
    out = _sc_attend(table, trow, idx3, w2r)
    return out + 0.0 * b2[0]
